# Initial kernel scaffold; baseline (speedup 1.0000x reference)
import jax
import jax.numpy as jnp
from jax.experimental import pallas as pl


def kernel(transformation, sig):
    raise NotImplementedError("write your pallas kernel here")


# speedup vs baseline: 1.0642x; 1.0642x over previous
import functools

import jax
import jax.numpy as jnp
from jax import lax
from jax.experimental import pallas as pl
from jax.experimental.pallas import tpu as pltpu
from jax.experimental.pallas import tpu_sc as plsc

B = 16
T_LEN = 4096
OUT_LEN = 2048
C = 128
NW = 32
ROWS_PER_W = (B * OUT_LEN) // NW
K = 128
N_CHUNKS = ROWS_PER_W // K
LANE = jnp.int32


def _bcast(vec, i):
    return jnp.full((16,), vec[i])


def _stn_body(ab_hbm, lin_hbm, sig_hbm, out_hbm,
              ab_v, lin_v, idx0_v, idx1_v, w0_v, w1_v,
              rows0_v, rows1_v, sem0, sem1):
    cid = lax.axis_index("c")
    sid = lax.axis_index("s")
    wid = cid * 16 + sid
    b = wid // 2
    jbase = (wid % 2) * ROWS_PER_W

    pltpu.sync_copy(ab_hbm.at[pl.ds(pl.multiple_of(wid * 32, 32), 32)], ab_v)
    a_vec = ab_v[pl.ds(0, 16)]
    c_vec = ab_v[pl.ds(16, 16)]
    row_base = b * T_LEN

    def chunk_fn(k, carry):
        j0 = pl.multiple_of(jbase + k * K, K)
        pltpu.sync_copy(lin_hbm.at[pl.ds(j0, K)], lin_v)

        def idx_fn(g, c2):
            s16 = pl.ds(g * 16, 16)
            lv = lin_v[s16]
            x = (a_vec * lv + c_vec) * jnp.float32(OUT_LEN)
            xt = x.astype(jnp.int32)
            xtf = xt.astype(jnp.float32)
            x0 = jnp.where(xtf > x, xt - 1, xt)
            x1 = x0 + 1
            x0c = jnp.clip(x0, 0, T_LEN - 1)
            x1c = jnp.clip(x1, 0, T_LEN - 1)
            idx0_v[s16] = row_base + x0c
            idx1_v[s16] = row_base + x1c
            w0_v[s16] = x1c.astype(jnp.float32) - x
            w1_v[s16] = x - x0c.astype(jnp.float32)
            return c2

        lax.fori_loop(0, K // 16, idx_fn, 0)

        cp0 = pltpu.async_copy(sig_hbm.at[idx0_v], rows0_v, sem0)
        cp1 = pltpu.async_copy(sig_hbm.at[idx1_v], rows1_v, sem1)
        cp0.wait()
        cp1.wait()

        def group_fn(g, c2):
            w0g = w0_v[pl.ds(g * 16, 16)]
            w1g = w1_v[pl.ds(g * 16, 16)]
            r0 = g * 16
            for i in range(16):
                w0b = _bcast(w0g, i)
                w1b = _bcast(w1g, i)
                r = r0 + i
                for sl in range(C // 16):
                    s = pl.ds(sl * 16, 16)
                    rows0_v[r, s] = w0b * rows0_v[r, s] + w1b * rows1_v[r, s]
            return c2

        lax.fori_loop(0, K // 16, group_fn, 0)

        out_r0 = pl.multiple_of(wid * ROWS_PER_W + k * K, K)
        pltpu.sync_copy(rows0_v, out_hbm.at[pl.ds(out_r0, K)])
        return carry

    lax.fori_loop(0, N_CHUNKS, chunk_fn, 0)


_stn_call = functools.partial(
    pl.kernel,
    mesh=plsc.VectorSubcoreMesh(core_axis_name="c", subcore_axis_name="s"),
    out_type=jax.ShapeDtypeStruct((B * OUT_LEN, C), jnp.float32),
    scratch_types=[
        pltpu.VMEM((32,), jnp.float32),
        pltpu.VMEM((K,), jnp.float32),
        pltpu.VMEM((K,), jnp.int32),
        pltpu.VMEM((K,), jnp.int32),
        pltpu.VMEM((K,), jnp.float32),
        pltpu.VMEM((K,), jnp.float32),
        pltpu.VMEM((K, C), jnp.float32),
        pltpu.VMEM((K, C), jnp.float32),
        pltpu.SemaphoreType.DMA,
        pltpu.SemaphoreType.DMA,
    ],
)(_stn_body)


def _round_bf16(v):
    u = lax.bitcast_convert_type(v.astype(jnp.float32), jnp.uint32)
    u = (u + jnp.uint32(0x7FFF) + ((u >> 16) & jnp.uint32(1))) & jnp.uint32(0xFFFF0000)
    return lax.bitcast_convert_type(u, jnp.float32)


def kernel(transformation, sig):
    lin = _round_bf16(jnp.linspace(0.0, 1.0, OUT_LEN, dtype=jnp.float32))
    transformation = _round_bf16(transformation)
    ab = jnp.repeat(transformation.astype(jnp.float32), 16, axis=0)
    ab = ab.reshape(B, 16, 2).transpose(0, 2, 1).reshape(-1)
    ab = ab.reshape(B, 32)
    ab = jnp.repeat(ab, 2, axis=0).reshape(-1)
    sig_flat = sig.reshape(B * T_LEN, C).astype(jnp.float32)
    out = _stn_call(ab, lin, sig_flat)
    return out.reshape(B, OUT_LEN, C)
